# Initial kernel scaffold; baseline (speedup 1.0000x reference)
import jax
import jax.numpy as jnp
from jax.experimental import pallas as pl


def kernel(query, reference_points, input_flatten, input_spatial_shapes, input_level_start_index, W_off, b_off, W_attn, b_attn, W_val, b_val, W_out, b_out):
    raise NotImplementedError("write your pallas kernel here")


# speedup vs baseline: 4476.9432x; 4476.9432x over previous
import functools

import numpy as np
import jax
import jax.numpy as jnp
from jax import lax
from jax.experimental import pallas as pl
from jax.experimental.pallas import tpu as pltpu
from jax.experimental.pallas import tpu_sc as plsc

DM = 192
HD = 6
DH = 32
NL = 4
NP = 4
_SPATIAL = np.array([[4, 48, 48], [4, 24, 24], [4, 12, 12], [4, 6, 6]], np.int64)
_SIZES = _SPATIAL[:, 0] * _SPATIAL[:, 1] * _SPATIAL[:, 2]
_STARTS = np.concatenate([[0], np.cumsum(_SIZES)[:-1]])
LEN_IN = int(_SIZES.sum())
LQ = 12240
NB = 2
NF = 2
G = NB * NF
NQ = G * LQ
HLP = HD * NL * NP
BQ = 240
NQB = LQ // BQ

_CW = np.zeros((8, HLP), np.float32)
for _h in range(HD):
    for _l in range(NL):
        for _p in range(NP):
            _c = _h * 16 + _l * 4 + _p
            _CW[0, _c] = _SPATIAL[_l, 2]
            _CW[1, _c] = _SPATIAL[_l, 1]
            _CW[2, _c] = _SPATIAL[_l, 0]
            _CW[3, _c] = _STARTS[_l]
            _CW[4, _c] = _h
_MBLK = np.kron(np.eye(HD, dtype=np.float32), np.ones((16, 16), np.float32))


def _prep_body(q_ref, if_ref, rx_ref, ry_ref, rz_ref,
               wx_ref, wy_ref, wz_ref, wa_ref, mblk_ref, wval_ref,
               bx_ref, by_ref, bz_ref, ba_ref, bval_ref, cw_ref,
               val_out, idx_out, w_out):
    g = pl.program_id(0)
    dn = (((1,), (1,)), ((), ()))

    vb = if_ref[0]
    val_out[0] = lax.dot_general(vb, wval_ref[...], dn,
                                 preferred_element_type=jnp.float32) + bval_ref[...]

    qb = q_ref[0]
    a = lax.dot_general(qb, wa_ref[...], dn,
                        preferred_element_type=jnp.float32) + ba_ref[...]
    m = jnp.max(a, axis=-1, keepdims=True)
    e = jnp.exp(a - m)
    den = lax.dot_general(e, mblk_ref[...], (((1,), (0,)), ((), ())),
                          preferred_element_type=jnp.float32)
    attn = e / jnp.maximum(den, 1e-30)

    wv = cw_ref[0:1, :]
    hv = cw_ref[1:2, :]
    dv = cw_ref[2:3, :]
    sv = cw_ref[3:4, :]
    hd = cw_ref[4:5, :]

    ox = lax.dot_general(qb, wx_ref[...], dn, preferred_element_type=jnp.float32) + bx_ref[...]
    oy = lax.dot_general(qb, wy_ref[...], dn, preferred_element_type=jnp.float32) + by_ref[...]
    oz = lax.dot_general(qb, wz_ref[...], dn, preferred_element_type=jnp.float32) + bz_ref[...]

    x = rx_ref[0] * wv + ox - 0.5
    y = ry_ref[0] * hv + oy - 0.5
    z = rz_ref[0] * dv + oz - 0.5

    cx = jnp.clip(jnp.floor(x), 0.0, wv - 2.0)
    cy = jnp.clip(jnp.floor(y), 0.0, hv - 2.0)
    cz = jnp.clip(jnp.floor(z), 0.0, dv - 2.0)
    wx0 = jnp.maximum(0.0, 1.0 - jnp.abs(x - cx))
    wx1 = jnp.maximum(0.0, 1.0 - jnp.abs(x - cx - 1.0))
    wy0 = jnp.maximum(0.0, 1.0 - jnp.abs(y - cy))
    wy1 = jnp.maximum(0.0, 1.0 - jnp.abs(y - cy - 1.0))
    wz0 = jnp.maximum(0.0, 1.0 - jnp.abs(z - cz))
    wz1 = jnp.maximum(0.0, 1.0 - jnp.abs(z - cz - 1.0))

    base = sv + (cz * hv + cy) * wv + cx + g.astype(jnp.float32) * float(LEN_IN)
    rowb = base * float(HD) + hd
    for c in range(8):
        dz, dy, dx = (c >> 2) & 1, (c >> 1) & 1, c & 1
        rc = rowb + ((dz * hv + dy) * wv + dx) * float(HD)
        wc = attn * (wz1 if dz else wz0) * (wy1 if dy else wy0) * (wx1 if dx else wx0)
        idx_out[0, c] = rc.astype(jnp.int32)
        w_out[0, c] = wc


def _run_prep(qf, iff, rx, ry, rz, wx, wy, wz, wa, wval,
              bx, by, bz, ba, bval):
    full = lambda s: pl.BlockSpec(s, lambda g, qb: (0,) * len(s))
    return pl.pallas_call(
        _prep_body,
        grid=(G, NQB),
        in_specs=[
            pl.BlockSpec((1, BQ, DM), lambda g, qb: (g, qb, 0)),
            pl.BlockSpec((1, BQ, DM), lambda g, qb: (g, qb, 0)),
            pl.BlockSpec((1, BQ, HLP), lambda g, qb: (g // NF, qb, 0)),
            pl.BlockSpec((1, BQ, HLP), lambda g, qb: (g // NF, qb, 0)),
            pl.BlockSpec((1, BQ, HLP), lambda g, qb: (g // NF, qb, 0)),
            full((HLP, DM)), full((HLP, DM)), full((HLP, DM)),
            full((HLP, DM)), full((HLP, HLP)), full((DM, DM)),
            full((1, HLP)), full((1, HLP)), full((1, HLP)),
            full((1, HLP)), full((1, DM)), full((8, HLP)),
        ],
        out_specs=[
            pl.BlockSpec((1, BQ, DM), lambda g, qb: (g, qb, 0)),
            pl.BlockSpec((1, 8, BQ, HLP), lambda g, qb: (g, 0, qb, 0)),
            pl.BlockSpec((1, 8, BQ, HLP), lambda g, qb: (g, 0, qb, 0)),
        ],
        out_shape=[
            jax.ShapeDtypeStruct((G, LEN_IN, DM), jnp.float32),
            jax.ShapeDtypeStruct((G, 8, LQ, HLP), jnp.int32),
            jax.ShapeDtypeStruct((G, 8, LQ, HLP), jnp.float32),
        ],
    )(qf, iff, rx, ry, rz, wx, wy, wz, wa, jnp.asarray(_MBLK), wval,
      bx, by, bz, ba, bval, jnp.asarray(_CW))


def _run_gather(table, idx, w):
    info = plsc.get_sparse_core_info()
    nc, ns = info.num_cores, info.num_subcores
    nw = nc * ns
    qpw = NQ // nw
    tpg = qpw and (LQ // qpw)
    mesh = plsc.VectorSubcoreMesh(core_axis_name="c", subcore_axis_name="s")

    @functools.partial(
        pl.kernel, mesh=mesh,
        out_type=jax.ShapeDtypeStruct((G, LQ, DM), jnp.float32),
        compiler_params=pltpu.CompilerParams(use_tc_tiling_on_sc=False),
        scratch_types=[
            pltpu.VMEM((8, HLP), jnp.int32),
            pltpu.VMEM((8, HLP), jnp.int32),
            pltpu.VMEM((8, HLP), jnp.float32),
            pltpu.VMEM((8, HLP), jnp.float32),
            pltpu.VMEM((8 * HLP, DH), jnp.float32),
            pltpu.VMEM((8 * HLP, DH), jnp.float32),
            pltpu.VMEM((DM,), jnp.float32),
            pltpu.SemaphoreType.DMA,
            pltpu.SemaphoreType.DMA,
        ],
    )
    def sc(table_hbm, idx_hbm, w_hbm, out_hbm,
           idx0, idx1, w0, w1, rows0, rows1, outv, sem0, sem1):
        wid = lax.axis_index("s") * nc + lax.axis_index("c")
        g = wid // tpg
        ql0 = (wid % tpg) * qpw

        def load_q(ql, idx_v, w_v):
            pltpu.sync_copy(idx_hbm.at[g, :, ql, :], idx_v)
            pltpu.sync_copy(w_hbm.at[g, :, ql, :], w_v)

        def fire(idx_v, rows_v, sem):
            for c in range(8):
                pltpu.async_copy(table_hbm.at[idx_v.at[c]],
                                 rows_v.at[pl.ds(c * HLP, HLP)], sem)

        def drain(idx_v, rows_v, sem):
            for c in range(8):
                pltpu.make_async_copy(table_hbm.at[idx_v.at[c]],
                                      rows_v.at[pl.ds(c * HLP, HLP)], sem).wait()

        def compute(rows_v, w_v, ql):
            for h in range(6):
                def body(c, acc):
                    a0, a1 = acc
                    wvec = w_v[c, pl.ds(h * 16, 16)]
                    for p in range(16):
                        j = c * HLP + h * 16 + p
                        wsc = wvec[p]
                        a0 = a0 + wsc * rows_v[j, pl.ds(0, 16)]
                        a1 = a1 + wsc * rows_v[j, pl.ds(16, 16)]
                    return (a0, a1)
                zero = jnp.zeros((16,), jnp.float32)
                a0, a1 = lax.fori_loop(0, 8, body, (zero, zero))
                outv[pl.ds(h * DH, 16)] = a0
                outv[pl.ds(h * DH + 16, 16)] = a1
            pltpu.sync_copy(outv, out_hbm.at[g, ql])

        load_q(ql0, idx0, w0)
        fire(idx0, rows0, sem0)

        def step(i2, carry):
            ql = ql0 + i2 * 2
            load_q(ql + 1, idx1, w1)
            fire(idx1, rows1, sem1)
            drain(idx0, rows0, sem0)
            compute(rows0, w0, ql)

            @pl.when(i2 * 2 + 2 < qpw)
            def _():
                load_q(ql + 2, idx0, w0)
                fire(idx0, rows0, sem0)

            drain(idx1, rows1, sem1)
            compute(rows1, w1, ql + 1)
            return carry

        lax.fori_loop(0, qpw // 2, step, 0)

    return sc(table, idx, w)


def _proj_body(x_ref, w_ref, b_ref, o_ref):
    o_ref[...] = lax.dot_general(x_ref[...], w_ref[...], (((1,), (1,)), ((), ())),
                                 preferred_element_type=jnp.float32) + b_ref[...]


def _run_out_proj(x, w, b):
    blk = 480
    return pl.pallas_call(
        _proj_body,
        grid=(NQ // blk,),
        in_specs=[
            pl.BlockSpec((blk, DM), lambda i: (i, 0)),
            pl.BlockSpec((DM, DM), lambda i: (0, 0)),
            pl.BlockSpec((1, DM), lambda i: (0, 0)),
        ],
        out_specs=pl.BlockSpec((blk, DM), lambda i: (i, 0)),
        out_shape=jax.ShapeDtypeStruct((NQ, DM), jnp.float32),
    )(x, w, b)


def kernel(query, reference_points, input_flatten, input_spatial_shapes,
           input_level_start_index, W_off, b_off, W_attn, b_attn,
           W_val, b_val, W_out, b_out):
    qf = query.reshape(G, LQ, DM)
    iff = input_flatten.reshape(G, LEN_IN, DM)

    wo = W_off.reshape(HD, NL, NP, 3, DM)
    bo = b_off.reshape(HD, NL, NP, 3)
    wx = wo[:, :, :, 0].reshape(HLP, DM)
    wy = wo[:, :, :, 1].reshape(HLP, DM)
    wz = wo[:, :, :, 2].reshape(HLP, DM)
    bx = bo[:, :, :, 0].reshape(1, HLP)
    by = bo[:, :, :, 1].reshape(1, HLP)
    bz = bo[:, :, :, 2].reshape(1, HLP)

    rp = reference_points
    rx = jnp.broadcast_to(rp[:, :, None, :, None, 0], (NB, LQ, HD, NL, NP)).reshape(NB, LQ, HLP)
    ry = jnp.broadcast_to(rp[:, :, None, :, None, 1], (NB, LQ, HD, NL, NP)).reshape(NB, LQ, HLP)
    rz = jnp.broadcast_to(rp[:, :, None, :, None, 2], (NB, LQ, HD, NL, NP)).reshape(NB, LQ, HLP)

    value, idx, w = _run_prep(qf, iff, rx, ry, rz, wx, wy, wz, W_attn, W_val,
                              bx, by, bz, b_attn.reshape(1, HLP), b_val.reshape(1, DM))

    table = value.reshape(G * LEN_IN * HD, DH)
    res = _run_gather(table, idx, w)

    out = _run_out_proj(res.reshape(NQ, DM), W_out, b_out.reshape(1, DM))
    return out.reshape(NB, NF, LQ, DM)
